# baseline (speedup 1.0000x reference)
import functools
import math

import jax
import jax.numpy as jnp
from jax.experimental import pallas as pl
from jax.experimental.pallas import tpu as pltpu

T = 2048
HID = 1024
H = 16
KV = 8
D = 64
E = 8
INTER = 2048
CAP = 30.0
THETA = 10000.0
EPS = 1e-5

BT = 256


def _rms_jnp(x, w):
    v = jnp.mean(jnp.square(x), axis=-1, keepdims=True)
    return x * jax.lax.rsqrt(v + EPS) * w


def _rms(x, w):
    v = jnp.mean(jnp.square(x), axis=-1, keepdims=True)
    return x * jax.lax.rsqrt(v + EPS) * w


def _k1_body(cos_ref, sin_ref, x_ref, wqkv_ref, qkv_ref):
    qkv = jnp.dot(x_ref[...], wqkv_ref[...], preferred_element_type=jnp.float32)
    half = D // 2
    cos = cos_ref[...]
    sin = sin_ref[...]
    pieces = []
    for c in range(H + KV):
        b = c * D
        x1 = qkv[:, b : b + half]
        x2 = qkv[:, b + half : b + D]
        pieces.append(x1 * cos - x2 * sin)
        pieces.append(x2 * cos + x1 * sin)
    pieces.append(qkv[:, (H + KV) * D :])
    qkv_ref[...] = jnp.concatenate(pieces, axis=1)


def _k1(positions, x, w_qkv):
    half = D // 2
    inv_freq = 1.0 / (THETA ** (jnp.arange(0, half, dtype=jnp.float32) / half))
    freqs = positions.astype(jnp.float32)[:, None] * inv_freq[None, :]
    cos = jnp.cos(freqs)
    sin = jnp.sin(freqs)
    return pl.pallas_call(
        _k1_body,
        grid=(T // BT,),
        in_specs=[
            pl.BlockSpec((BT, half), lambda i: (i, 0)),
            pl.BlockSpec((BT, half), lambda i: (i, 0)),
            pl.BlockSpec((BT, HID), lambda i: (i, 0)),
            pl.BlockSpec((HID, (H + 2 * KV) * D), lambda i: (0, 0)),
        ],
        out_specs=pl.BlockSpec((BT, (H + 2 * KV) * D), lambda i: (i, 0)),
        out_shape=jax.ShapeDtypeStruct((T, (H + 2 * KV) * D), jnp.float32),
    )(cos, sin, x, w_qkv)


def _k2_body(q_ref, k_ref, v_ref, o_ref):
    i = pl.program_id(1)
    q = q_ref[0]
    k = k_ref[0]
    v = v_ref[0]
    s = jax.lax.dot_general(q, k, (((1,), (1,)), ((), ())),
                            preferred_element_type=jnp.float32)
    s = s * (D ** -0.5)
    s = CAP * jnp.tanh(s / CAP)
    rows = i * BT + jax.lax.broadcasted_iota(jnp.int32, (BT, T), 0)
    cols = jax.lax.broadcasted_iota(jnp.int32, (BT, T), 1)
    s = jnp.where(cols <= rows, s, -1e30)
    m = jnp.max(s, axis=-1, keepdims=True)
    p = jnp.exp(s - m)
    acc = p[:, 0:128]
    for c in range(1, T // 128):
        acc = acc + p[:, c * 128:(c + 1) * 128]
    w = 128
    while w > 1:
        w //= 2
        acc = acc[:, :w] + acc[:, w:2 * w]
    p = p / acc
    o_ref[0] = jnp.dot(p, v, preferred_element_type=jnp.float32)


def _k2(q3, k3, v3):
    return pl.pallas_call(
        _k2_body,
        grid=(H, T // BT),
        in_specs=[
            pl.BlockSpec((1, BT, D), lambda h, i: (h, i, 0)),
            pl.BlockSpec((1, T, D), lambda h, i: (h // (H // KV), 0, 0)),
            pl.BlockSpec((1, T, D), lambda h, i: (h // (H // KV), 0, 0)),
        ],
        out_specs=pl.BlockSpec((1, BT, D), lambda h, i: (h, i, 0)),
        out_shape=jax.ShapeDtypeStruct((H, T, D), jnp.float32),
    )(q3, k3, v3)


def _k3_body(attn_ref, wo_ref, out_ref):
    out_ref[...] = jnp.dot(attn_ref[...], wo_ref[...],
                           preferred_element_type=jnp.float32)


def _k3(attn, w_o):
    return pl.pallas_call(
        _k3_body,
        grid=(T // BT,),
        in_specs=[
            pl.BlockSpec((BT, H * D), lambda i: (i, 0)),
            pl.BlockSpec((H * D, HID), lambda i: (0, 0)),
        ],
        out_specs=pl.BlockSpec((BT, HID), lambda i: (i, 0)),
        out_shape=jax.ShapeDtypeStruct((T, HID), jnp.float32),
    )(attn, w_o)


IC = 1024
NJ = INTER // IC


def _k4_body(x2_ref, comb_ref, wg_ref, wu_ref, wd_ref, h_ref, ln_ref, out_ref, acc_ref):
    e = pl.program_id(1)
    j = pl.program_id(2)
    x2 = x2_ref[...]
    g = jnp.dot(x2, wg_ref[0], preferred_element_type=jnp.float32)
    u = jnp.dot(x2, wu_ref[0], preferred_element_type=jnp.float32)
    a = jax.nn.gelu(g, approximate=True) * u
    y = jnp.dot(a, wd_ref[0], preferred_element_type=jnp.float32)
    eh = jax.lax.broadcasted_iota(jnp.int32, (E, 1), 0) == e
    ce = jnp.dot(comb_ref[...], eh.astype(jnp.float32),
                 preferred_element_type=jnp.float32)
    contrib = y * ce

    @pl.when((e == 0) & (j == 0))
    def _():
        acc_ref[...] = contrib

    @pl.when((e > 0) | (j > 0))
    def _():
        acc_ref[...] += contrib

    @pl.when((e == E - 1) & (j == NJ - 1))
    def _():
        out_ref[...] = h_ref[...] + _rms(acc_ref[...], ln_ref[...])


def _k4(x2, comb, w_gate_up, w_down, h, ln_post_moe):
    return pl.pallas_call(
        _k4_body,
        grid=(T // BT, E, NJ),
        in_specs=[
            pl.BlockSpec((BT, HID), lambda i, e, j: (i, 0)),
            pl.BlockSpec((BT, E), lambda i, e, j: (i, 0)),
            pl.BlockSpec((1, HID, IC), lambda i, e, j: (e, 0, j)),
            pl.BlockSpec((1, HID, IC), lambda i, e, j: (e, 0, j + NJ)),
            pl.BlockSpec((1, IC, HID), lambda i, e, j: (e, j, 0)),
            pl.BlockSpec((BT, HID), lambda i, e, j: (i, 0)),
            pl.BlockSpec((1, HID), lambda i, e, j: (0, 0)),
        ],
        out_specs=pl.BlockSpec((BT, HID), lambda i, e, j: (i, 0)),
        out_shape=jax.ShapeDtypeStruct((T, HID), jnp.float32),
        scratch_shapes=[pltpu.VMEM((BT, HID), jnp.float32)],
    )(x2, comb, w_gate_up, w_gate_up, w_down, h, ln_post_moe.reshape(1, HID))


def kernel(positions, hidden_states, w_qkv, w_o, w_gate, w_gate_up, w_down,
           ln_pre_attn, ln_post_attn, ln_pre_moe, ln_post_moe):
    x = _rms_jnp(hidden_states, ln_pre_attn)
    qkv = _k1(positions, x, w_qkv)
    q3 = qkv[:, : H * D].reshape(T, H, D).transpose(1, 0, 2)
    k3 = qkv[:, H * D : (H + KV) * D].reshape(T, KV, D).transpose(1, 0, 2)
    v3 = qkv[:, (H + KV) * D :].reshape(T, KV, D).transpose(1, 0, 2)
    attn = _k2(q3, k3, v3).transpose(1, 0, 2).reshape(T, H * D)
    attn_out = _k3(attn, w_o)
    h = hidden_states + _rms_jnp(attn_out, ln_post_attn)
    x2 = _rms_jnp(h, ln_pre_moe)
    rl = x2 @ w_gate
    rl = jnp.tanh(rl / CAP) * CAP
    rprobs = jax.nn.softmax(rl, axis=-1)
    topw, topi = jax.lax.top_k(rprobs, 2)
    comb = jnp.sum(jax.nn.one_hot(topi, E, dtype=rprobs.dtype) * topw[..., None], axis=1)
    out = _k4(x2, comb, w_gate_up, w_down, h, ln_post_moe)
    return out


# speedup vs baseline: 1.3322x; 1.3322x over previous
import functools
import math

import jax
import jax.numpy as jnp
from jax import lax
from jax.experimental import pallas as pl
from jax.experimental.pallas import tpu as pltpu
from jax.experimental.pallas import tpu_sc as plsc

T = 2048
HID = 1024
H = 16
KV = 8
D = 64
E = 8
INTER = 2048
CAP = 30.0
THETA = 10000.0
EPS = 1e-5

BT = 256


def _rms_jnp(x, w):
    v = jnp.mean(jnp.square(x), axis=-1, keepdims=True)
    return x * jax.lax.rsqrt(v + EPS) * w


def _rms(x, w):
    v = jnp.mean(jnp.square(x), axis=-1, keepdims=True)
    return x * jax.lax.rsqrt(v + EPS) * w


def _k1_body(cos_ref, sin_ref, x_ref, wqkv_ref, qkv_ref):
    qkv = jnp.dot(x_ref[...], wqkv_ref[...], preferred_element_type=jnp.float32)
    half = D // 2
    cos = cos_ref[...]
    sin = sin_ref[...]
    pieces = []
    for c in range(H + KV):
        b = c * D
        x1 = qkv[:, b : b + half]
        x2 = qkv[:, b + half : b + D]
        pieces.append(x1 * cos - x2 * sin)
        pieces.append(x2 * cos + x1 * sin)
    pieces.append(qkv[:, (H + KV) * D :])
    qkv_ref[...] = jnp.concatenate(pieces, axis=1)


def _k1(positions, x, w_qkv):
    half = D // 2
    inv_freq = 1.0 / (THETA ** (jnp.arange(0, half, dtype=jnp.float32) / half))
    freqs = positions.astype(jnp.float32)[:, None] * inv_freq[None, :]
    cos = jnp.cos(freqs)
    sin = jnp.sin(freqs)
    return pl.pallas_call(
        _k1_body,
        grid=(T // BT,),
        in_specs=[
            pl.BlockSpec((BT, half), lambda i: (i, 0)),
            pl.BlockSpec((BT, half), lambda i: (i, 0)),
            pl.BlockSpec((BT, HID), lambda i: (i, 0)),
            pl.BlockSpec((HID, (H + 2 * KV) * D), lambda i: (0, 0)),
        ],
        out_specs=pl.BlockSpec((BT, (H + 2 * KV) * D), lambda i: (i, 0)),
        out_shape=jax.ShapeDtypeStruct((T, (H + 2 * KV) * D), jnp.float32),
    )(cos, sin, x, w_qkv)


def _k2_body(q_ref, k_ref, v_ref, o_ref):
    i = pl.program_id(1)
    q = q_ref[0]
    k = k_ref[0]
    v = v_ref[0]
    s = jax.lax.dot_general(q, k, (((1,), (1,)), ((), ())),
                            preferred_element_type=jnp.float32)
    s = s * (D ** -0.5)
    s = CAP * jnp.tanh(s / CAP)
    rows = i * BT + jax.lax.broadcasted_iota(jnp.int32, (BT, T), 0)
    cols = jax.lax.broadcasted_iota(jnp.int32, (BT, T), 1)
    s = jnp.where(cols <= rows, s, -1e30)
    m = jnp.max(s, axis=-1, keepdims=True)
    p = jnp.exp(s - m)
    acc = p[:, 0:128]
    for c in range(1, T // 128):
        acc = acc + p[:, c * 128:(c + 1) * 128]
    w = 128
    while w > 1:
        w //= 2
        acc = acc[:, :w] + acc[:, w:2 * w]
    p = p / acc
    o_ref[0] = jnp.dot(p, v, preferred_element_type=jnp.float32)


def _k2(q3, k3, v3):
    return pl.pallas_call(
        _k2_body,
        grid=(H, T // BT),
        in_specs=[
            pl.BlockSpec((1, BT, D), lambda h, i: (h, i, 0)),
            pl.BlockSpec((1, T, D), lambda h, i: (h // (H // KV), 0, 0)),
            pl.BlockSpec((1, T, D), lambda h, i: (h // (H // KV), 0, 0)),
        ],
        out_specs=pl.BlockSpec((1, BT, D), lambda h, i: (h, i, 0)),
        out_shape=jax.ShapeDtypeStruct((H, T, D), jnp.float32),
    )(q3, k3, v3)


def _k3_body(attn_ref, wo_ref, out_ref):
    out_ref[...] = jnp.dot(attn_ref[...], wo_ref[...],
                           preferred_element_type=jnp.float32)


def _k3(attn, w_o):
    return pl.pallas_call(
        _k3_body,
        grid=(T // BT,),
        in_specs=[
            pl.BlockSpec((BT, H * D), lambda i: (i, 0)),
            pl.BlockSpec((H * D, HID), lambda i: (0, 0)),
        ],
        out_specs=pl.BlockSpec((BT, HID), lambda i: (i, 0)),
        out_shape=jax.ShapeDtypeStruct((T, HID), jnp.float32),
    )(attn, w_o)


IC = 1024
NJ = INTER // IC


def _k4_body(x2_ref, comb_ref, wg_ref, wu_ref, wd_ref, h_ref, ln_ref, out_ref, acc_ref):
    e = pl.program_id(1)
    j = pl.program_id(2)
    x2 = x2_ref[...]
    g = jnp.dot(x2, wg_ref[0], preferred_element_type=jnp.float32)
    u = jnp.dot(x2, wu_ref[0], preferred_element_type=jnp.float32)
    a = jax.nn.gelu(g, approximate=True) * u
    y = jnp.dot(a, wd_ref[0], preferred_element_type=jnp.float32)
    eh = jax.lax.broadcasted_iota(jnp.int32, (E, 1), 0) == e
    ce = jnp.dot(comb_ref[...], eh.astype(jnp.float32),
                 preferred_element_type=jnp.float32)
    contrib = y * ce

    @pl.when((e == 0) & (j == 0))
    def _():
        acc_ref[...] = contrib

    @pl.when((e > 0) | (j > 0))
    def _():
        acc_ref[...] += contrib

    @pl.when((e == E - 1) & (j == NJ - 1))
    def _():
        out_ref[...] = h_ref[...] + _rms(acc_ref[...], ln_ref[...])


def _k4(x2, comb, w_gate_up, w_down, h, ln_post_moe):
    return pl.pallas_call(
        _k4_body,
        grid=(T // BT, E, NJ),
        in_specs=[
            pl.BlockSpec((BT, HID), lambda i, e, j: (i, 0)),
            pl.BlockSpec((BT, E), lambda i, e, j: (i, 0)),
            pl.BlockSpec((1, HID, IC), lambda i, e, j: (e, 0, j)),
            pl.BlockSpec((1, HID, IC), lambda i, e, j: (e, 0, j + NJ)),
            pl.BlockSpec((1, IC, HID), lambda i, e, j: (e, j, 0)),
            pl.BlockSpec((BT, HID), lambda i, e, j: (i, 0)),
            pl.BlockSpec((1, HID), lambda i, e, j: (0, 0)),
        ],
        out_specs=pl.BlockSpec((BT, HID), lambda i, e, j: (i, 0)),
        out_shape=jax.ShapeDtypeStruct((T, HID), jnp.float32),
        scratch_shapes=[pltpu.VMEM((BT, HID), jnp.float32)],
    )(x2, comb, w_gate_up, w_gate_up, w_down, h, ln_post_moe.reshape(1, HID))


PB = 256
NP = 2 * T
NT = NP // PB + E - 1
ROWS = NT * PB
CSB = 512


def _route_body(topi_ref, dst_ref, texp_ref):
    e_iota = jax.lax.broadcasted_iota(jnp.int32, (1, E), 1)
    oh1 = (topi_ref[:, 0:1] == e_iota).astype(jnp.float32)
    oh2 = (topi_ref[:, 1:2] == e_iota).astype(jnp.float32)
    oh = jnp.concatenate([oh1, oh2], axis=0)
    r = jax.lax.broadcasted_iota(jnp.int32, (CSB, CSB), 0)
    c = jax.lax.broadcasted_iota(jnp.int32, (CSB, CSB), 1)
    L = (r >= c).astype(jnp.float32)
    carry = jnp.zeros((1, E), jnp.float32)
    ranks = []
    for b in range(NP // CSB):
        ob = oh[b * CSB : (b + 1) * CSB]
        cs = jnp.dot(L, ob, preferred_element_type=jnp.float32) + carry
        ranks.append(jnp.sum(cs * ob, axis=-1, keepdims=True))
        carry = carry + jnp.sum(ob, axis=0, keepdims=True)
    rank = jnp.concatenate(ranks, axis=0)
    counts = carry
    te = jnp.floor((counts + (PB - 1)) / PB)
    m8 = (jax.lax.broadcasted_iota(jnp.int32, (E, E), 0)
          <= jax.lax.broadcasted_iota(jnp.int32, (E, E), 1)).astype(jnp.float32)
    cumt = jnp.dot(te, m8, preferred_element_type=jnp.float32)
    pstart = (cumt - te) * PB
    pstart_col = jax.lax.dot_general(
        (jax.lax.broadcasted_iota(jnp.int32, (E, E), 0)
         == jax.lax.broadcasted_iota(jnp.int32, (E, E), 1)).astype(jnp.float32),
        pstart, (((1,), (1,)), ((), ())), preferred_element_type=jnp.float32)
    base = jnp.dot(oh, pstart_col, preferred_element_type=jnp.float32)
    dst_ref[...] = (base + rank - 1.0).astype(jnp.int32)
    cumt_col = jax.lax.dot_general(
        (jax.lax.broadcasted_iota(jnp.int32, (E, E), 0)
         == jax.lax.broadcasted_iota(jnp.int32, (E, E), 1)).astype(jnp.float32),
        cumt, (((1,), (1,)), ((), ())), preferred_element_type=jnp.float32)
    g_iota = jax.lax.broadcasted_iota(jnp.int32, (1, NT), 1).astype(jnp.float32)
    texp_ref[...] = jnp.sum(
        (g_iota >= cumt_col).astype(jnp.float32), axis=0, keepdims=True
    ).astype(jnp.int32)


def _route(topi):
    return pl.pallas_call(
        _route_body,
        grid=(1,),
        in_specs=[pl.BlockSpec((T, 2), lambda i: (0, 0))],
        out_specs=[
            pl.BlockSpec((NP, 1), lambda i: (0, 0)),
            pl.BlockSpec((1, NT), lambda i: (0, 0)),
        ],
        out_shape=[
            jax.ShapeDtypeStruct((NP, 1), jnp.int32),
            jax.ShapeDtypeStruct((1, NT), jnp.int32),
        ],
    )(topi)


_NC, _NS = 2, 16
NW = _NC * _NS
PPW = NP // NW
SUB = 32
NSUB = PPW // SUB


def _sc_scatter(x2, dstm):
    @functools.partial(
        pl.kernel,
        mesh=plsc.VectorSubcoreMesh(core_axis_name="c", subcore_axis_name="s"),
        out_type=jax.ShapeDtypeStruct((ROWS, HID), jnp.float32),
        scratch_types=[
            pltpu.VMEM((NSUB, SUB), jnp.int32),
            pltpu.VMEM((SUB, HID), jnp.float32),
            pltpu.SemaphoreType.DMA,
        ],
    )
    def k(x2_hbm, idx_hbm, xs_hbm, idx_v, rows_v, sem):
        wid = lax.axis_index("s") * _NC + lax.axis_index("c")
        pltpu.sync_copy(idx_hbm.at[wid], idx_v)
        src0 = (wid % (T // PPW)) * PPW
        for j in range(NSUB):
            pltpu.sync_copy(x2_hbm.at[pl.ds(src0 + j * SUB, SUB)], rows_v)
            pltpu.async_copy(rows_v, xs_hbm.at[idx_v.at[j]], sem).wait()

    return k(x2, dstm)


def _sc_gather(ys, dstm):
    @functools.partial(
        pl.kernel,
        mesh=plsc.VectorSubcoreMesh(core_axis_name="c", subcore_axis_name="s"),
        out_type=jax.ShapeDtypeStruct((NP, HID), jnp.float32),
        scratch_types=[
            pltpu.VMEM((NSUB, SUB), jnp.int32),
            pltpu.VMEM((SUB, HID), jnp.float32),
            pltpu.SemaphoreType.DMA,
        ],
    )
    def k(ys_hbm, idx_hbm, out_hbm, idx_v, rows_v, sem):
        wid = lax.axis_index("s") * _NC + lax.axis_index("c")
        pltpu.sync_copy(idx_hbm.at[wid], idx_v)
        dst0 = wid * PPW
        for j in range(NSUB):
            pltpu.async_copy(ys_hbm.at[idx_v.at[j]], rows_v, sem).wait()
            pltpu.sync_copy(rows_v, out_hbm.at[pl.ds(dst0 + j * SUB, SUB)])

    return k(ys, dstm)


def _gmm_body(texp_ref, xs_ref, wg_ref, wu_ref, wd_ref, ys_ref):
    j = pl.program_id(1)
    xs = xs_ref[...]
    g = jnp.dot(xs, wg_ref[0], preferred_element_type=jnp.float32)
    u = jnp.dot(xs, wu_ref[0], preferred_element_type=jnp.float32)
    a = jax.nn.gelu(g, approximate=True) * u
    y = jnp.dot(a, wd_ref[0], preferred_element_type=jnp.float32)

    @pl.when(j == 0)
    def _():
        ys_ref[...] = y

    @pl.when(j > 0)
    def _():
        ys_ref[...] += y


def _gmm(texp, xs, w_gate_up, w_down):
    return pl.pallas_call(
        _gmm_body,
        grid_spec=pltpu.PrefetchScalarGridSpec(
            num_scalar_prefetch=1,
            grid=(NT, NJ),
            in_specs=[
                pl.BlockSpec((PB, HID), lambda g, j, t: (g, 0)),
                pl.BlockSpec((1, HID, IC), lambda g, j, t: (t[g], 0, j)),
                pl.BlockSpec((1, HID, IC), lambda g, j, t: (t[g], 0, j + NJ)),
                pl.BlockSpec((1, IC, HID), lambda g, j, t: (t[g], j, 0)),
            ],
            out_specs=pl.BlockSpec((PB, HID), lambda g, j, t: (g, 0)),
        ),
        out_shape=jax.ShapeDtypeStruct((ROWS, HID), jnp.float32),
    )(texp, xs, w_gate_up, w_gate_up, w_down)


def _k5_body(h_ref, m1_ref, m2_ref, w_ref, ln_ref, out_ref):
    w1 = w_ref[:, 0:1]
    w2 = w_ref[:, 1:2]
    moe = w1 * m1_ref[...] + w2 * m2_ref[...]
    out_ref[...] = h_ref[...] + _rms(moe, ln_ref[...])


def _k5(h, moe1, moe2, topw, ln_post_moe):
    return pl.pallas_call(
        _k5_body,
        grid=(T // BT,),
        in_specs=[
            pl.BlockSpec((BT, HID), lambda i: (i, 0)),
            pl.BlockSpec((BT, HID), lambda i: (i, 0)),
            pl.BlockSpec((BT, HID), lambda i: (i, 0)),
            pl.BlockSpec((BT, 2), lambda i: (i, 0)),
            pl.BlockSpec((1, HID), lambda i: (0, 0)),
        ],
        out_specs=pl.BlockSpec((BT, HID), lambda i: (i, 0)),
        out_shape=jax.ShapeDtypeStruct((T, HID), jnp.float32),
    )(h, moe1, moe2, topw, ln_post_moe.reshape(1, HID))


def kernel(positions, hidden_states, w_qkv, w_o, w_gate, w_gate_up, w_down,
           ln_pre_attn, ln_post_attn, ln_pre_moe, ln_post_moe):
    x = _rms_jnp(hidden_states, ln_pre_attn)
    qkv = _k1(positions, x, w_qkv)
    q3 = qkv[:, : H * D].reshape(T, H, D).transpose(1, 0, 2)
    k3 = qkv[:, H * D : (H + KV) * D].reshape(T, KV, D).transpose(1, 0, 2)
    v3 = qkv[:, (H + KV) * D :].reshape(T, KV, D).transpose(1, 0, 2)
    attn = _k2(q3, k3, v3).transpose(1, 0, 2).reshape(T, H * D)
    attn_out = _k3(attn, w_o)
    h = hidden_states + _rms_jnp(attn_out, ln_post_attn)
    x2 = _rms_jnp(h, ln_pre_moe)
    rl = x2 @ w_gate
    rl = jnp.tanh(rl / CAP) * CAP
    rprobs = jax.nn.softmax(rl, axis=-1)
    topw, topi = jax.lax.top_k(rprobs, 2)
    dst, texp = _route(topi.astype(jnp.int32))
    dstm = dst.reshape(NW, NSUB, SUB)
    xs = _sc_scatter(x2, dstm)
    ys = _gmm(texp.reshape(NT), xs, w_gate_up, w_down)
    pairs = _sc_gather(ys, dstm)
    out = _k5(h, pairs[:T], pairs[T:], topw, ln_post_moe)
    return out
